# Initial kernel scaffold; baseline (speedup 1.0000x reference)
import functools

import jax
import jax.numpy as jnp
from jax import lax
from jax.experimental import pallas as pl
from jax.experimental.pallas import tpu as pltpu
from jax.experimental.pallas import tpu_sc as plsc

N = 100000
E = 3200000
NUM_GRAPHS = 512
HID = 20

NC, NS = 2, 16
NW = NC * NS
SPT = 782
EPAD = NW * SPT * 128
KCH = 17
NGRP = SPT // KCH
NPAD = N + 16
RINIT = NPAD // NS
ROUT = N // NS

PPAD = 102400
PSPT = 25
PK = 5
PNG = PSPT // PK
GPAD = NUM_GRAPHS + 16
GINIT = GPAD // NS
GOUT = NUM_GRAPHS // NS

_mesh = plsc.VectorSubcoreMesh(
    core_axis_name="c", subcore_axis_name="s", num_cores=NC, num_subcores=NS
)


def _make_scatter_sum(d, spt, kch, ngrp, acc_rows, rinit, rout, out_rows):
  @functools.partial(
      pl.kernel,
      out_type=(
          jax.ShapeDtypeStruct((out_rows, d), jnp.float32),
          jax.ShapeDtypeStruct((out_rows, d), jnp.float32),
      ),
      mesh=_mesh,
      scratch_types=[
          pltpu.VMEM_SHARED((acc_rows, d), jnp.float32),
          pltpu.VMEM((kch, 128), jnp.int32),
          pltpu.VMEM((kch, 128), jnp.int32),
          pltpu.VMEM((kch, 128, d), jnp.float32),
          pltpu.SemaphoreType.DMA,
          pltpu.SemaphoreType.DMA,
      ],
  )
  def scatter_sum(h, srcr, dstr, zr, out0, out1, acc, idxs, idxd, rows,
                  gsem, ssem):
    cid = lax.axis_index("c")
    sid = lax.axis_index("s")
    wid = sid * NC + cid

    pltpu.sync_copy(zr.at[pl.ds(sid * rinit, rinit)],
                    acc.at[pl.ds(sid * rinit, rinit)])
    plsc.subcore_barrier()

    def group(g, carry):
      base = wid * spt + g * kch
      pltpu.sync_copy(srcr.at[pl.ds(base, kch)], idxs)
      pltpu.sync_copy(dstr.at[pl.ds(base, kch)], idxd)
      gcps = [
          pltpu.async_copy(h.at[idxs.at[j]], rows.at[j], gsem)
          for j in range(kch)
      ]
      for cp in gcps:
        cp.wait()
      scps = [
          pltpu.async_copy(rows.at[j], acc.at[idxd.at[j]], ssem, add=True)
          for j in range(kch)
      ]
      for cp in scps:
        cp.wait()
      return carry

    lax.fori_loop(0, ngrp, group, 0)
    plsc.subcore_barrier()

    @pl.when(cid == 0)
    def _():
      pltpu.sync_copy(acc.at[pl.ds(sid * rout, rout)],
                      out0.at[pl.ds(sid * rout, rout)])

    @pl.when(cid == 1)
    def _():
      pltpu.sync_copy(acc.at[pl.ds(sid * rout, rout)],
                      out1.at[pl.ds(sid * rout, rout)])

  return scatter_sum


RB = 2000
NBLK = N // RB


def _mlp(h, p0, p1, w1, b1, w2, b2, d):
  def body(h_ref, p0_ref, p1_ref, w1_ref, b1_ref, w2_ref, b2_ref,
           z2_ref, st_ref):
    z = h_ref[...] + p0_ref[...] + p1_ref[...]
    a = jnp.maximum(
        jnp.dot(z, w1_ref[...], preferred_element_type=jnp.float32)
        + b1_ref[...], 0.0)
    z2 = jnp.dot(a, w2_ref[...], preferred_element_type=jnp.float32) \
        + b2_ref[...]
    z2_ref[...] = z2

    @pl.when(pl.program_id(0) == 0)
    def _():
      st_ref[...] = jnp.zeros_like(st_ref)

    s1 = jnp.sum(z2, axis=0, keepdims=True)
    s2 = jnp.sum(z2 * z2, axis=0, keepdims=True)
    st_ref[...] += jnp.concatenate([s1, s2], axis=0)

  return pl.pallas_call(
      body,
      grid=(NBLK,),
      in_specs=[
          pl.BlockSpec((RB, d), lambda i: (i, 0)),
          pl.BlockSpec((RB, d), lambda i: (i, 0)),
          pl.BlockSpec((RB, d), lambda i: (i, 0)),
          pl.BlockSpec((d, HID), lambda i: (0, 0)),
          pl.BlockSpec((1, HID), lambda i: (0, 0)),
          pl.BlockSpec((HID, HID), lambda i: (0, 0)),
          pl.BlockSpec((1, HID), lambda i: (0, 0)),
      ],
      out_specs=[
          pl.BlockSpec((RB, HID), lambda i: (i, 0)),
          pl.BlockSpec((2, HID), lambda i: (0, 0)),
      ],
      out_shape=[
          jax.ShapeDtypeStruct((N, HID), jnp.float32),
          jax.ShapeDtypeStruct((2, HID), jnp.float32),
      ],
  )(h, p0, p1, w1, b1, w2, b2)


def _bnrelu(z2, st, g, b):
  def body(z2_ref, st_ref, g_ref, b_ref, o_ref):
    st = st_ref[...]
    m = st[0:1, :] * (1.0 / N)
    v = st[1:2, :] * (1.0 / N) - m * m
    scale = g_ref[...] * lax.rsqrt(v + 1e-5)
    shift = b_ref[...] - m * scale
    o_ref[...] = jnp.maximum(z2_ref[...] * scale + shift, 0.0)

  return pl.pallas_call(
      body,
      grid=(NBLK,),
      in_specs=[
          pl.BlockSpec((RB, HID), lambda i: (i, 0)),
          pl.BlockSpec((2, HID), lambda i: (0, 0)),
          pl.BlockSpec((1, HID), lambda i: (0, 0)),
          pl.BlockSpec((1, HID), lambda i: (0, 0)),
      ],
      out_specs=pl.BlockSpec((RB, HID), lambda i: (i, 0)),
      out_shape=jax.ShapeDtypeStruct((N, HID), jnp.float32),
  )(z2, st, g, b)


def _fc(q0, q1, w, b):
  def body(q0_ref, q1_ref, w_ref, b_ref, o_ref):
    o_ref[...] = jnp.dot(q0_ref[...] + q1_ref[...], w_ref[...],
                         preferred_element_type=jnp.float32) + b_ref[...]

  return pl.pallas_call(
      body,
      out_shape=jax.ShapeDtypeStruct((NUM_GRAPHS, 2), jnp.float32),
  )(q0, q1, w, b)


def kernel(x, edge_index, batch,
           c0_W1, c0_b1, c0_W2, c0_b2,
           c1_W1, c1_b1, c1_W2, c1_b2,
           c2_W1, c2_b1, c2_W2, c2_b2,
           bn0_g, bn0_b, bn1_g, bn1_b, bn2_g, bn2_b,
           fc_W, fc_b):
  src = edge_index[0]
  dst = edge_index[1]
  epad = EPAD - E
  srcr = jnp.concatenate([src, jnp.zeros((epad,), jnp.int32)]
                         ).reshape(EPAD // 128, 128)
  dstr = jnp.concatenate([dst, jnp.full((epad,), N, jnp.int32)]
                         ).reshape(EPAD // 128, 128)
  z10 = jnp.zeros((NPAD, 10), jnp.float32)
  z20 = jnp.zeros((NPAD, HID), jnp.float32)

  agg10 = _make_scatter_sum(10, SPT, KCH, NGRP, NPAD, RINIT, ROUT, N)
  agg20 = _make_scatter_sum(HID, SPT, KCH, NGRP, NPAD, RINIT, ROUT, N)

  layers = [
      (c0_W1, c0_b1, c0_W2, c0_b2, bn0_g, bn0_b, 10, agg10, z10),
      (c1_W1, c1_b1, c1_W2, c1_b2, bn1_g, bn1_b, HID, agg20, z20),
      (c2_W1, c2_b1, c2_W2, c2_b2, bn2_g, bn2_b, HID, agg20, z20),
  ]
  h = x
  for (w1, b1, w2, b2, g, bb, d, aggk, zr) in layers:
    p0, p1 = aggk(h, srcr, dstr, zr)
    z2, st = _mlp(h, p0, p1, w1, b1.reshape(1, HID), w2, b2.reshape(1, HID), d)
    h = _bnrelu(z2, st, g.reshape(1, HID), bb.reshape(1, HID))

  ppad = PPAD - N
  ridx = jnp.concatenate([jnp.arange(N, dtype=jnp.int32),
                          jnp.zeros((ppad,), jnp.int32)]
                         ).reshape(PPAD // 128, 128)
  bidx = jnp.concatenate([batch, jnp.full((ppad,), NUM_GRAPHS, jnp.int32)]
                         ).reshape(PPAD // 128, 128)
  zg = jnp.zeros((GPAD, HID), jnp.float32)
  poolk = _make_scatter_sum(HID, PSPT, PK, PNG, GPAD, GINIT, GOUT, NUM_GRAPHS)
  q0, q1 = poolk(h, ridx, bidx, zg)
  return _fc(q0, q1, fc_W, fc_b.reshape(1, 2))


# speedup vs baseline: 17.5091x; 17.5091x over previous
import functools

import jax
import jax.numpy as jnp
from jax import lax
from jax.experimental import pallas as pl
from jax.experimental.pallas import tpu as pltpu
from jax.experimental.pallas import tpu_sc as plsc

N = 100000
E = 3200000
NUM_GRAPHS = 512
HID = 20
DR = 10
DH = 16

NC, NS = 2, 16
NW = NC * NS
SPT = 784
EPAD = NW * SPT * 128
KCH = 8
NGRP = SPT // KCH
NPAD = N + 96
RINIT = NPAD // NS

PPAD = 131072
PSPT = 32
PK = 8
PNG = PSPT // PK
GPAD = 640
GINIT = GPAD // NS

def _make_scatter_sum(spt, kch, ngrp, acc_rows, rinit):
  @functools.partial(
      pl.kernel,
      out_type=(
          jax.ShapeDtypeStruct((acc_rows, DH), jnp.float32),
          jax.ShapeDtypeStruct((acc_rows, DH), jnp.float32),
      ),
      mesh=plsc.VectorSubcoreMesh(core_axis_name="c", subcore_axis_name="s",
                                  num_cores=NC, num_subcores=NS),
      compiler_params=pltpu.CompilerParams(use_tc_tiling_on_sc=False),
      scratch_types=[
          pltpu.VMEM_SHARED((acc_rows, DH), jnp.float32),
          pltpu.VMEM((kch, 128), jnp.int32),
          pltpu.VMEM((kch, 128), jnp.int32),
          pltpu.VMEM((kch, 128, DH), jnp.float32),
          pltpu.SemaphoreType.DMA,
          pltpu.SemaphoreType.DMA,
      ],
  )
  def scatter_sum(h, srcr, dstr, zr, out0, out1, acc, idxs, idxd, rows,
                  gsem, ssem):
    cid = lax.axis_index("c")
    sid = lax.axis_index("s")
    wid = sid * NC + cid

    pltpu.sync_copy(zr.at[pl.ds(sid * rinit, rinit)],
                    acc.at[pl.ds(sid * rinit, rinit)])
    plsc.subcore_barrier()

    def group(g, carry):
      base = wid * spt + g * kch
      pltpu.sync_copy(srcr.at[pl.ds(base, kch)], idxs)
      pltpu.sync_copy(dstr.at[pl.ds(base, kch)], idxd)
      gcps = [
          pltpu.async_copy(h.at[idxs.at[j]], rows.at[j], gsem)
          for j in range(kch)
      ]
      for cp in gcps:
        cp.wait()
      scps = [
          pltpu.async_copy(rows.at[j], acc.at[idxd.at[j]], ssem, add=True)
          for j in range(kch)
      ]
      for cp in scps:
        cp.wait()
      return carry

    lax.fori_loop(0, ngrp, group, 0)
    plsc.subcore_barrier()

    @pl.when(cid == 0)
    def _():
      pltpu.sync_copy(acc.at[pl.ds(sid * rinit, rinit)],
                      out0.at[pl.ds(sid * rinit, rinit)])

    @pl.when(cid == 1)
    def _():
      pltpu.sync_copy(acc.at[pl.ds(sid * rinit, rinit)],
                      out1.at[pl.ds(sid * rinit, rinit)])

  return scatter_sum


RB = 2000
NBLK = N // RB


def _mlp(parts, w1, b1, w2, b2):
  nh = len(parts)

  def body(*refs):
    part_refs = refs[:3 * nh]
    w1_ref, b1_ref, w2_ref, b2_ref, z2_ref, st_ref = refs[3 * nh:]
    halves = [
        part_refs[3 * i][...] + part_refs[3 * i + 1][...]
        + part_refs[3 * i + 2][...]
        for i in range(nh)
    ]
    z = halves[0] if nh == 1 else jnp.concatenate(halves, axis=1)
    a = jnp.maximum(
        jnp.dot(z, w1_ref[...], preferred_element_type=jnp.float32)
        + b1_ref[...], 0.0)
    z2 = jnp.dot(a, w2_ref[...], preferred_element_type=jnp.float32) \
        + b2_ref[...]
    z2_ref[...] = z2

    @pl.when(pl.program_id(0) == 0)
    def _():
      st_ref[...] = jnp.zeros_like(st_ref)

    s1 = jnp.sum(z2, axis=0, keepdims=True)
    s2 = jnp.sum(z2 * z2, axis=0, keepdims=True)
    st_ref[...] += jnp.concatenate([s1, s2], axis=0)

  din = nh * DH
  flat = [a for t in parts for a in t]
  return pl.pallas_call(
      body,
      grid=(NBLK,),
      in_specs=[pl.BlockSpec((RB, DH), lambda i: (i, 0))] * (3 * nh) + [
          pl.BlockSpec((din, HID), lambda i: (0, 0)),
          pl.BlockSpec((1, HID), lambda i: (0, 0)),
          pl.BlockSpec((HID, HID), lambda i: (0, 0)),
          pl.BlockSpec((1, HID), lambda i: (0, 0)),
      ],
      out_specs=[
          pl.BlockSpec((RB, HID), lambda i: (i, 0)),
          pl.BlockSpec((2, HID), lambda i: (0, 0)),
      ],
      out_shape=[
          jax.ShapeDtypeStruct((N, HID), jnp.float32),
          jax.ShapeDtypeStruct((2, HID), jnp.float32),
      ],
  )(*flat, w1, b1, w2, b2)


def _bnrelu(z2, st, g, b):
  def body(z2_ref, st_ref, g_ref, b_ref, oa_ref, ob_ref):
    st = st_ref[...]
    m = st[0:1, :] * (1.0 / N)
    v = st[1:2, :] * (1.0 / N) - m * m
    scale = g_ref[...] * lax.rsqrt(v + 1e-5)
    shift = b_ref[...] - m * scale
    res = jnp.maximum(z2_ref[...] * scale + shift, 0.0)
    zpad = jnp.zeros((res.shape[0], DH - DR), jnp.float32)
    oa_ref[...] = jnp.concatenate([res[:, 0:DR], zpad], axis=1)
    ob_ref[...] = jnp.concatenate([res[:, DR:HID], zpad], axis=1)

  return pl.pallas_call(
      body,
      grid=(NBLK,),
      in_specs=[
          pl.BlockSpec((RB, HID), lambda i: (i, 0)),
          pl.BlockSpec((2, HID), lambda i: (0, 0)),
          pl.BlockSpec((1, HID), lambda i: (0, 0)),
          pl.BlockSpec((1, HID), lambda i: (0, 0)),
      ],
      out_specs=[
          pl.BlockSpec((RB, DH), lambda i: (i, 0)),
          pl.BlockSpec((RB, DH), lambda i: (i, 0)),
      ],
      out_shape=[
          jax.ShapeDtypeStruct((N, DH), jnp.float32),
          jax.ShapeDtypeStruct((N, DH), jnp.float32),
      ],
  )(z2, st, g, b)


def _fc(qa0, qa1, qb0, qb1, w, b):
  def body(qa0_ref, qa1_ref, qb0_ref, qb1_ref, w_ref, b_ref, o_ref):
    q = jnp.concatenate([qa0_ref[...] + qa1_ref[...],
                         qb0_ref[...] + qb1_ref[...]], axis=1)
    o_ref[...] = jnp.dot(q, w_ref[...],
                         preferred_element_type=jnp.float32) + b_ref[...]

  return pl.pallas_call(
      body,
      grid=(1,),
      in_specs=[pl.BlockSpec((NUM_GRAPHS, DH), lambda i: (0, 0))] * 4 + [
          pl.BlockSpec((2 * DH, 2), lambda i: (0, 0)),
          pl.BlockSpec((1, 2), lambda i: (0, 0)),
      ],
      out_specs=pl.BlockSpec((NUM_GRAPHS, 2), lambda i: (0, 0)),
      out_shape=jax.ShapeDtypeStruct((NUM_GRAPHS, 2), jnp.float32),
  )(qa0, qa1, qb0, qb1, w, b)


def kernel(x, edge_index, batch,
           c0_W1, c0_b1, c0_W2, c0_b2,
           c1_W1, c1_b1, c1_W2, c1_b2,
           c2_W1, c2_b1, c2_W2, c2_b2,
           bn0_g, bn0_b, bn1_g, bn1_b, bn2_g, bn2_b,
           fc_W, fc_b):
  src = edge_index[0]
  dst = edge_index[1]
  epad = EPAD - E
  srcr = jnp.concatenate([src, jnp.zeros((epad,), jnp.int32)]
                         ).reshape(EPAD // 128, 128)
  dstr = jnp.concatenate([dst, jnp.full((epad,), N, jnp.int32)]
                         ).reshape(EPAD // 128, 128)
  zn = jnp.zeros((NPAD, DH), jnp.float32)

  agg = _make_scatter_sum(SPT, KCH, NGRP, NPAD, RINIT)

  zw = jnp.zeros((DH - DR, HID), jnp.float32)
  c0_W1p = jnp.concatenate([c0_W1, zw], axis=0)
  c1_W1p = jnp.concatenate([c1_W1[:DR], zw, c1_W1[DR:], zw], axis=0)
  c2_W1p = jnp.concatenate([c2_W1[:DR], zw, c2_W1[DR:], zw], axis=0)
  zf = jnp.zeros((DH - DR, 2), jnp.float32)
  fc_Wp = jnp.concatenate([fc_W[:DR], zf, fc_W[DR:], zf], axis=0)

  xp = jnp.pad(x, ((0, 0), (0, DH - DR)))
  p0, p1 = agg(xp, srcr, dstr, zn)
  z2, st = _mlp([(xp, p0, p1)], c0_W1p, c0_b1.reshape(1, HID),
                c0_W2, c0_b2.reshape(1, HID))
  ha, hb = _bnrelu(z2, st, bn0_g.reshape(1, HID), bn0_b.reshape(1, HID))

  for (w1, b1, w2, b2, g, bb) in (
      (c1_W1p, c1_b1, c1_W2, c1_b2, bn1_g, bn1_b),
      (c2_W1p, c2_b1, c2_W2, c2_b2, bn2_g, bn2_b),
  ):
    p0a, p1a = agg(ha, srcr, dstr, zn)
    p0b, p1b = agg(hb, srcr, dstr, zn)
    z2, st = _mlp([(ha, p0a, p1a), (hb, p0b, p1b)],
                  w1, b1.reshape(1, HID), w2, b2.reshape(1, HID))
    ha, hb = _bnrelu(z2, st, g.reshape(1, HID), bb.reshape(1, HID))

  ppad = PPAD - N
  ridx = jnp.concatenate([jnp.arange(N, dtype=jnp.int32),
                          jnp.zeros((ppad,), jnp.int32)]
                         ).reshape(PPAD // 128, 128)
  bidx = jnp.concatenate([batch, jnp.full((ppad,), NUM_GRAPHS, jnp.int32)]
                         ).reshape(PPAD // 128, 128)
  zg = jnp.zeros((GPAD, DH), jnp.float32)
  poolk = _make_scatter_sum(PSPT, PK, PNG, GPAD, GINIT)
  qa0, qa1 = poolk(ha, ridx, bidx, zg)
  qb0, qb1 = poolk(hb, ridx, bidx, zg)
  return _fc(qa0, qa1, qb0, qb1, fc_Wp, fc_b.reshape(1, 2))


# speedup vs baseline: 22.2114x; 1.2686x over previous
import functools

import jax
import jax.numpy as jnp
from jax import lax
from jax.experimental import pallas as pl
from jax.experimental.pallas import tpu as pltpu
from jax.experimental.pallas import tpu_sc as plsc

N = 100000
E = 3200000
NUM_GRAPHS = 512
HID = 20
DR = 10
DH = 16

NC, NS = 2, 16
NW = NC * NS
CH = 512
EPAD = 3211264
EPT1 = EPAD // NW
EPT2 = EPAD // NS
NPAD = N + 96
RINIT = NPAD // NS

PPAD = 131072
GPAD = 640
GINIT = GPAD // NS


def _make_agg(split_edges, ept, acc_rows, rinit):
  nch = ept // CH
  nt = nch // 2

  @functools.partial(
      pl.kernel,
      out_type=(
          jax.ShapeDtypeStruct((acc_rows, DH), jnp.float32),
          jax.ShapeDtypeStruct((acc_rows, DH), jnp.float32),
      ),
      mesh=plsc.VectorSubcoreMesh(core_axis_name="c", subcore_axis_name="s",
                                  num_cores=NC, num_subcores=NS),
      compiler_params=pltpu.CompilerParams(use_tc_tiling_on_sc=False),
      scratch_types=[
          pltpu.VMEM_SHARED((acc_rows, DH), jnp.float32),
          pltpu.VMEM((CH,), jnp.int32),
          pltpu.VMEM((CH,), jnp.int32),
          pltpu.VMEM((CH,), jnp.int32),
          pltpu.VMEM((CH,), jnp.int32),
          pltpu.VMEM((CH, DH), jnp.float32),
          pltpu.VMEM((CH, DH), jnp.float32),
          pltpu.SemaphoreType.DMA,
          pltpu.SemaphoreType.DMA,
          pltpu.SemaphoreType.DMA,
          pltpu.SemaphoreType.DMA,
      ],
  )
  def agg(ta, tb, ia, ib, srcf, dstf, oa, ob, acc, sv0, dv0, sv1, dv1,
          r0, r1, g0, g1, s0, s1):
    cid = lax.axis_index("c")
    sid = lax.axis_index("s")
    base = (sid * NC + cid) * ept if split_edges else sid * ept

    def run(table, initr, out):
      pltpu.sync_copy(initr.at[pl.ds(sid * rinit, rinit)],
                      acc.at[pl.ds(sid * rinit, rinit)])
      plsc.subcore_barrier()

      pltpu.sync_copy(srcf.at[pl.ds(base, CH)], sv0)
      pltpu.sync_copy(dstf.at[pl.ds(base, CH)], dv0)
      pltpu.async_copy(table.at[sv0], r0, g0)

      def pair(t, carry):
        off = base + 2 * t * CH

        @pl.when(t > 0)
        def _():
          pltpu.make_async_copy(table.at[pl.ds(0, CH)], r1, s1).wait()

        pltpu.sync_copy(srcf.at[pl.ds(off + CH, CH)], sv1)
        pltpu.sync_copy(dstf.at[pl.ds(off + CH, CH)], dv1)
        pltpu.async_copy(table.at[sv1], r1, g1)
        pltpu.make_async_copy(table.at[pl.ds(0, CH)], r0, g0).wait()
        pltpu.async_copy(r0, acc.at[dv0], s0, add=True)
        pltpu.make_async_copy(table.at[pl.ds(0, CH)], r0, s0).wait()

        @pl.when(t + 1 < nt)
        def _():
          pltpu.sync_copy(srcf.at[pl.ds(off + 2 * CH, CH)], sv0)
          pltpu.sync_copy(dstf.at[pl.ds(off + 2 * CH, CH)], dv0)
          pltpu.async_copy(table.at[sv0], r0, g0)

        pltpu.make_async_copy(table.at[pl.ds(0, CH)], r1, g1).wait()
        pltpu.async_copy(r1, acc.at[dv1], s1, add=True)
        return carry

      lax.fori_loop(0, nt, pair, 0)
      pltpu.make_async_copy(table.at[pl.ds(0, CH)], r1, s1).wait()
      plsc.subcore_barrier()
      pltpu.sync_copy(acc.at[pl.ds(sid * rinit, rinit)],
                      out.at[pl.ds(sid * rinit, rinit)])

    @pl.when(cid == 0)
    def _():
      run(ta, ia, oa)

    @pl.when(cid == 1)
    def _():
      run(tb, ib, ob)

  return agg


RB = 2000
NBLK = N // RB


def _mlp(groups, w1, b1, w2, b2):
  sizes = [len(g) for g in groups]
  ng = len(groups)
  nin = sum(sizes)

  def body(*refs):
    in_refs = refs[:nin]
    w1_ref, b1_ref, w2_ref, b2_ref, z2_ref, st_ref = refs[nin:]
    parts = []
    k = 0
    for sz in sizes:
      acc = in_refs[k][...]
      for j in range(1, sz):
        acc = acc + in_refs[k + j][...]
      parts.append(acc)
      k += sz
    z = parts[0] if ng == 1 else jnp.concatenate(parts, axis=1)
    a = jnp.maximum(
        jnp.dot(z, w1_ref[...], preferred_element_type=jnp.float32)
        + b1_ref[...], 0.0)
    z2 = jnp.dot(a, w2_ref[...], preferred_element_type=jnp.float32) \
        + b2_ref[...]
    z2_ref[...] = z2

    @pl.when(pl.program_id(0) == 0)
    def _():
      st_ref[...] = jnp.zeros_like(st_ref)

    s1 = jnp.sum(z2, axis=0, keepdims=True)
    s2 = jnp.sum(z2 * z2, axis=0, keepdims=True)
    st_ref[...] += jnp.concatenate([s1, s2], axis=0)

  din = ng * DH
  flat = [a for g in groups for a in g]
  return pl.pallas_call(
      body,
      grid=(NBLK,),
      in_specs=[pl.BlockSpec((RB, DH), lambda i: (i, 0))] * nin + [
          pl.BlockSpec((din, HID), lambda i: (0, 0)),
          pl.BlockSpec((1, HID), lambda i: (0, 0)),
          pl.BlockSpec((HID, HID), lambda i: (0, 0)),
          pl.BlockSpec((1, HID), lambda i: (0, 0)),
      ],
      out_specs=[
          pl.BlockSpec((RB, HID), lambda i: (i, 0)),
          pl.BlockSpec((2, HID), lambda i: (0, 0)),
      ],
      out_shape=[
          jax.ShapeDtypeStruct((N, HID), jnp.float32),
          jax.ShapeDtypeStruct((2, HID), jnp.float32),
      ],
  )(*flat, w1, b1, w2, b2)


def _bnrelu(z2, st, g, b):
  def body(z2_ref, st_ref, g_ref, b_ref, oa_ref, ob_ref):
    st = st_ref[...]
    m = st[0:1, :] * (1.0 / N)
    v = st[1:2, :] * (1.0 / N) - m * m
    scale = g_ref[...] * lax.rsqrt(v + 1e-5)
    shift = b_ref[...] - m * scale
    res = jnp.maximum(z2_ref[...] * scale + shift, 0.0)
    zpad = jnp.zeros((res.shape[0], DH - DR), jnp.float32)
    oa_ref[...] = jnp.concatenate([res[:, 0:DR], zpad], axis=1)
    ob_ref[...] = jnp.concatenate([res[:, DR:HID], zpad], axis=1)

  return pl.pallas_call(
      body,
      grid=(NBLK,),
      in_specs=[
          pl.BlockSpec((RB, HID), lambda i: (i, 0)),
          pl.BlockSpec((2, HID), lambda i: (0, 0)),
          pl.BlockSpec((1, HID), lambda i: (0, 0)),
          pl.BlockSpec((1, HID), lambda i: (0, 0)),
      ],
      out_specs=[
          pl.BlockSpec((RB, DH), lambda i: (i, 0)),
          pl.BlockSpec((RB, DH), lambda i: (i, 0)),
      ],
      out_shape=[
          jax.ShapeDtypeStruct((NPAD, DH), jnp.float32),
          jax.ShapeDtypeStruct((NPAD, DH), jnp.float32),
      ],
  )(z2, st, g, b)


def _fc(qa, qb, w, b):
  def body(qa_ref, qb_ref, w_ref, b_ref, o_ref):
    q = jnp.concatenate([qa_ref[...], qb_ref[...]], axis=1)
    o_ref[...] = jnp.dot(q, w_ref[...],
                         preferred_element_type=jnp.float32) + b_ref[...]

  return pl.pallas_call(
      body,
      grid=(1,),
      in_specs=[pl.BlockSpec((NUM_GRAPHS, DH), lambda i: (0, 0))] * 2 + [
          pl.BlockSpec((2 * DH, 2), lambda i: (0, 0)),
          pl.BlockSpec((1, 2), lambda i: (0, 0)),
      ],
      out_specs=pl.BlockSpec((NUM_GRAPHS, 2), lambda i: (0, 0)),
      out_shape=jax.ShapeDtypeStruct((NUM_GRAPHS, 2), jnp.float32),
  )(qa, qb, w, b)


def kernel(x, edge_index, batch,
           c0_W1, c0_b1, c0_W2, c0_b2,
           c1_W1, c1_b1, c1_W2, c1_b2,
           c2_W1, c2_b1, c2_W2, c2_b2,
           bn0_g, bn0_b, bn1_g, bn1_b, bn2_g, bn2_b,
           fc_W, fc_b):
  src = edge_index[0]
  dst = edge_index[1]
  epad = EPAD - E
  srcf = jnp.concatenate([src, jnp.zeros((epad,), jnp.int32)])
  dstf = jnp.concatenate([dst, jnp.full((epad,), N, jnp.int32)])
  zn = jnp.zeros((NPAD, DH), jnp.float32)

  zw = jnp.zeros((DH - DR, HID), jnp.float32)
  c0_W1p = jnp.concatenate([c0_W1, zw], axis=0)
  c1_W1p = jnp.concatenate([c1_W1[:DR], zw, c1_W1[DR:], zw], axis=0)
  c2_W1p = jnp.concatenate([c2_W1[:DR], zw, c2_W1[DR:], zw], axis=0)
  zf = jnp.zeros((DH - DR, 2), jnp.float32)
  fc_Wp = jnp.concatenate([fc_W[:DR], zf, fc_W[DR:], zf], axis=0)

  agg1 = _make_agg(True, EPT1, NPAD, RINIT)
  agg2 = _make_agg(False, EPT2, NPAD, RINIT)

  xp = jnp.pad(x, ((0, NPAD - N), (0, DH - DR)))
  p0, p1 = agg1(xp, xp, xp, zn, srcf, dstf)
  z2, st = _mlp([[p0, p1]], c0_W1p, c0_b1.reshape(1, HID),
                c0_W2, c0_b2.reshape(1, HID))
  ha, hb = _bnrelu(z2, st, bn0_g.reshape(1, HID), bn0_b.reshape(1, HID))

  for (w1, b1, w2, b2, g, bb) in (
      (c1_W1p, c1_b1, c1_W2, c1_b2, bn1_g, bn1_b),
      (c2_W1p, c2_b1, c2_W2, c2_b2, bn2_g, bn2_b),
  ):
    pa, pb = agg2(ha, hb, ha, hb, srcf, dstf)
    z2, st = _mlp([[pa], [pb]], w1, b1.reshape(1, HID),
                  w2, b2.reshape(1, HID))
    ha, hb = _bnrelu(z2, st, g.reshape(1, HID), bb.reshape(1, HID))

  ppad = PPAD - N
  ridx = jnp.concatenate([jnp.arange(N, dtype=jnp.int32),
                          jnp.zeros((ppad,), jnp.int32)])
  bidx = jnp.concatenate([batch, jnp.full((ppad,), NUM_GRAPHS, jnp.int32)])
  zg = jnp.zeros((GPAD, DH), jnp.float32)
  poolk = _make_agg(False, PPAD // NS, GPAD, GINIT)
  qa, qb = poolk(ha, hb, zg, zg, ridx, bidx)
  return _fc(qa[:NUM_GRAPHS], qb[:NUM_GRAPHS], fc_Wp, fc_b.reshape(1, 2))
